# baseline (speedup 1.0000x reference)
import functools

import jax
import jax.numpy as jnp
from jax import lax
from jax.experimental import pallas as pl
from jax.experimental.pallas import tpu as pltpu
from jax.experimental.pallas import tpu_sc as plsc

VOCAB = 8192
NC = 2
NS = 16
NW = NC * NS
K = 4
NBUF = 3
LOOKAHEAD = 3


@functools.partial(jax.jit, static_argnames=())
def kernel(x, table):
    b, s = x.shape
    total = b * s
    per_w = total // NW
    nchunk = per_w // K
    idx3 = x.reshape(NW, nchunk, K)

    mesh = plsc.VectorSubcoreMesh(
        core_axis_name="c", subcore_axis_name="s",
        num_cores=NC, num_subcores=NS,
    )

    @functools.partial(
        pl.kernel,
        mesh=mesh,
        out_type=jax.ShapeDtypeStruct((total, VOCAB), jnp.float32),
        scratch_types=[
            pltpu.VMEM((nchunk, K), jnp.int32),
            pltpu.VMEM((NBUF, K, VOCAB), jnp.float32),
            pltpu.SemaphoreType.DMA((NBUF,)),
            pltpu.SemaphoreType.DMA((NBUF,)),
        ],
    )
    def gather_kernel(idx_hbm, table_hbm, out_hbm, idx_v, buf_v, gsem, ssem):
        wid = lax.axis_index("s") * NC + lax.axis_index("c")
        base = wid * per_w
        pltpu.sync_copy(idx_hbm.at[wid], idx_v)

        def g_copy(c, bslot):
            return pltpu.make_async_copy(
                table_hbm.at[idx_v.at[c]], buf_v.at[bslot], gsem.at[bslot])

        def s_copy(c, bslot):
            return pltpu.make_async_copy(
                buf_v.at[bslot], out_hbm.at[pl.ds(base + c * K, K)],
                ssem.at[bslot])

        for c in range(LOOKAHEAD):
            g_copy(c, c % NBUF).start()

        nc_main = (nchunk // NBUF) * NBUF
        if nc_main == nchunk:
            nc_main -= NBUF

        def step(c, j):
            g_copy(c, j).wait()
            s_copy(c, j).start()
            nxt = c + LOOKAHEAD
            jn = (j + LOOKAHEAD) % NBUF

            def prefetch():
                @pl.when(nxt >= NBUF)
                def _():
                    s_copy(nxt - NBUF, jn).wait()
                g_copy(nxt, jn).start()

            if isinstance(c, int):
                if nxt < nchunk:
                    prefetch()
            else:
                pl.when(nxt < nchunk)(prefetch)

        @pl.loop(0, nc_main, step=NBUF)
        def _(c0):
            for j in range(NBUF):
                step(c0 + j, j)

        for c in range(nc_main, nchunk):
            step(c, c % NBUF)
        for c in range(nchunk - NBUF, nchunk):
            s_copy(c, c % NBUF).wait()

    out = gather_kernel(idx3, table)
    return out.reshape(b, s, VOCAB)


# speedup vs baseline: 1.0033x; 1.0033x over previous
import functools

import jax
import jax.numpy as jnp
from jax import lax
from jax.experimental import pallas as pl
from jax.experimental.pallas import tpu as pltpu
from jax.experimental.pallas import tpu_sc as plsc

VOCAB = 8192
NC = 2
NS = 16
NW = NC * NS
K = 4
NBUF = 3
LOOKAHEAD = 2


@functools.partial(jax.jit, static_argnames=())
def kernel(x, table):
    b, s = x.shape
    total = b * s
    per_w = total // NW
    nchunk = per_w // K
    idx3 = x.reshape(NW, nchunk, K)

    mesh = plsc.VectorSubcoreMesh(
        core_axis_name="c", subcore_axis_name="s",
        num_cores=NC, num_subcores=NS,
    )

    @functools.partial(
        pl.kernel,
        mesh=mesh,
        out_type=jax.ShapeDtypeStruct((total, VOCAB), jnp.float32),
        scratch_types=[
            pltpu.VMEM((nchunk, K), jnp.int32),
            pltpu.VMEM((NBUF, K, VOCAB), jnp.float32),
            pltpu.SemaphoreType.DMA((NBUF,)),
            pltpu.SemaphoreType.DMA((NBUF,)),
        ],
    )
    def gather_kernel(idx_hbm, table_hbm, out_hbm, idx_v, buf_v, gsem, ssem):
        wid = lax.axis_index("s") * NC + lax.axis_index("c")
        base = wid * per_w
        pltpu.sync_copy(idx_hbm.at[wid], idx_v)

        def g_copy(c, bslot):
            return pltpu.make_async_copy(
                table_hbm.at[idx_v.at[c]], buf_v.at[bslot], gsem.at[bslot])

        def s_copy(c, bslot):
            return pltpu.make_async_copy(
                buf_v.at[bslot], out_hbm.at[pl.ds(base + c * K, K)],
                ssem.at[bslot])

        for c in range(LOOKAHEAD):
            g_copy(c, c % NBUF).start()

        nc_main = (nchunk // NBUF) * NBUF
        if nc_main == nchunk:
            nc_main -= NBUF

        def step(c, j):
            g_copy(c, j).wait()
            s_copy(c, j).start()
            nxt = c + LOOKAHEAD
            jn = (j + LOOKAHEAD) % NBUF

            def prefetch():
                @pl.when(nxt >= NBUF)
                def _():
                    s_copy(nxt - NBUF, jn).wait()
                g_copy(nxt, jn).start()

            if isinstance(c, int):
                if nxt < nchunk:
                    prefetch()
            else:
                pl.when(nxt < nchunk)(prefetch)

        @pl.loop(0, nc_main, step=NBUF)
        def _(c0):
            for j in range(NBUF):
                step(c0 + j, j)

        for c in range(nc_main, nchunk):
            step(c, c % NBUF)
        for c in range(nchunk - NBUF, nchunk):
            s_copy(c, c % NBUF).wait()

    out = gather_kernel(idx3, table)
    return out.reshape(b, s, VOCAB)
